# Initial kernel scaffold; baseline (speedup 1.0000x reference)
import jax
import jax.numpy as jnp
from jax.experimental import pallas as pl


def kernel(x, edge_index, graph_ids, W_fc, attn_l, attn_r, W1, b1, W2, b2):
    raise NotImplementedError("write your pallas kernel here")


# speedup vs baseline: 10.3011x; 10.3011x over previous
import jax
import jax.numpy as jnp
from jax import lax
from jax.experimental import pallas as pl
from jax.experimental.pallas import tpu as pltpu
from jax.experimental.pallas import tpu_sc as plsc

N = 10000
E = 320000
D = 128
H = 5
F = 128
G = 16
CLS = 10
NEG_SLOPE = 0.2

FX = F + 16

NCORE = 2
NTILE = 16
C = 128
NCHUNK = -(-E // (NTILE * C))
EPT = NCHUNK * C
E_PAD = EPT * NTILE
N_PAD = 10112
ROWS_PT = N_PAD // NTILE


_BN = 1000


def _a_body(x_ref, w_ref, al_ref, ar_ref, featx_ref, er2_ref):
    xb = x_ref[...]
    w = w_ref[0]
    f = lax.dot_general(xb, w, (((1,), (1,)), ((), ())),
                        preferred_element_type=jnp.float32)
    al = al_ref[0, 0]
    ar = ar_ref[0, 0]
    el = (f * al[None, :]).sum(axis=1)
    er = (f * ar[None, :]).sum(axis=1)
    lane = lax.broadcasted_iota(jnp.int32, (_BN, 16), 1)
    featx_ref[0] = jnp.concatenate(
        [f, jnp.where(lane == 0, el[:, None], 0.0)], axis=1)
    er2_ref[0] = jnp.where(lane == 0, er[:, None], 0.0)


def _stage_a(x, w_r, al3, ar3):
    return pl.pallas_call(
        _a_body,
        grid=(N // _BN, H),
        in_specs=[
            pl.BlockSpec((_BN, D), lambda i, h: (i, 0)),
            pl.BlockSpec((1, F, D), lambda i, h: (h, 0, 0)),
            pl.BlockSpec((1, 1, F), lambda i, h: (h, 0, 0)),
            pl.BlockSpec((1, 1, F), lambda i, h: (h, 0, 0)),
        ],
        out_specs=[
            pl.BlockSpec((1, _BN, FX), lambda i, h: (h, i, 0)),
            pl.BlockSpec((1, _BN, 16), lambda i, h: (h, i, 0)),
        ],
        out_shape=[
            jax.ShapeDtypeStruct((H, N, FX), jnp.float32),
            jax.ShapeDtypeStruct((H, N, 16), jnp.float32),
        ],
    )(x, w_r, al3, ar3)


def _b_body(featx_hbm, er2_hbm, src_hbm, dst_hbm, zrows_hbm, acc_out,
            srcidx, dstbuf, dstidx, fbuf, erg, acc_sh):
    c = lax.axis_index("c")
    s = lax.axis_index("s")
    tbase = s * EPT
    rows0 = s * ROWS_PT

    for hi in range(3):
        h = c + 2 * hi

        @pl.when(h < H)
        def _head():
            pltpu.sync_copy(zrows_hbm, acc_sh.at[pl.ds(rows0, ROWS_PT)])
            plsc.subcore_barrier()
            hN = h * N

            def _chunk(k, _):
                base = tbase + k * C
                pltpu.sync_copy(src_hbm.at[pl.ds(base, C)], srcidx)
                pltpu.sync_copy(dst_hbm.at[pl.ds(base, C)], dstbuf)
                for j in range(C // 16):
                    off = pl.ds(j * 16, 16)
                    srcidx[off] = srcidx[off] + hN
                    dstidx[off] = dstbuf[off] + hN
                pltpu.sync_copy(featx_hbm.at[srcidx], fbuf)
                pltpu.sync_copy(er2_hbm.at[dstidx], erg)

                lane = lax.iota(jnp.int32, 16)

                def _edge16(jj, _):
                    for ll in range(16):
                        cc = jj * 16 + ll
                        ev = fbuf[cc, pl.ds(F, 16)] + erg[cc, :]
                        ev = jnp.where(ev > 0, ev, NEG_SLOPE * ev)
                        ezv = jnp.exp(ev)
                        ezc = jnp.where(base + cc < E, ezv[0], 0.0)
                        for j in range(F // 16):
                            off2 = pl.ds(j * 16, 16)
                            fbuf[cc, off2] = fbuf[cc, off2] * ezc
                        fbuf[cc, pl.ds(F, 16)] = jnp.where(lane == 0, ezc,
                                                           0.0)
                    return 0

                lax.fori_loop(0, C // 16, _edge16, 0)
                pltpu.sync_copy(fbuf, acc_sh.at[dstbuf], add=True)
                return 0

            lax.fori_loop(0, NCHUNK, _chunk, 0)
            plsc.subcore_barrier()
            pltpu.sync_copy(acc_sh.at[pl.ds(rows0, ROWS_PT)],
                            acc_out.at[h].at[pl.ds(rows0, ROWS_PT)])
            plsc.subcore_barrier()


_stage_b = pl.kernel(
    _b_body,
    out_type=jax.ShapeDtypeStruct((H, N_PAD, FX), jnp.float32),
    mesh=plsc.VectorSubcoreMesh(core_axis_name="c", subcore_axis_name="s"),
    compiler_params=pltpu.CompilerParams(needs_layout_passes=False,
                                         use_tc_tiling_on_sc=False),
    scratch_types=[
        pltpu.VMEM((C,), jnp.int32),
        pltpu.VMEM((C,), jnp.int32),
        pltpu.VMEM((C,), jnp.int32),
        pltpu.VMEM((C, FX), jnp.float32),
        pltpu.VMEM((C, 16), jnp.float32),
        pltpu.VMEM_SHARED((N_PAD, FX), jnp.float32),
    ],
)


_BC = 632


def _c_body(acc_ref, gid_ref, w1_ref, b1_ref, w2_ref, b2_ref,
            out_ref, hg_ref, cnt_ref):
    i = pl.program_id(0)

    @pl.when(i == 0)
    def _():
        hg_ref[...] = jnp.zeros_like(hg_ref)
        cnt_ref[...] = jnp.zeros_like(cnt_ref)

    full = acc_ref[...]
    a = full[:, :, :F]
    sden = full[:, :, F]
    r = jnp.maximum(a / jnp.maximum(sden, 1e-30)[:, :, None], 0.0)
    hi = r.sum(axis=0) * (1.0 / H)
    g = gid_ref[0, 0, :]
    m = (g[None, :] == lax.broadcasted_iota(jnp.int32, (G, _BC), 0))
    m = m.astype(jnp.float32)
    hg_ref[...] += lax.dot_general(m, hi, (((1,), (0,)), ((), ())),
                                   preferred_element_type=jnp.float32)
    cnt_ref[...] += jnp.broadcast_to(m.sum(axis=1)[:, None], (G, F))

    @pl.when(i == pl.num_programs(0) - 1)
    def _():
        hg = hg_ref[...] / jnp.maximum(cnt_ref[...], 1.0)
        h1 = jnp.maximum(
            lax.dot_general(hg, w1_ref[...], (((1,), (0,)), ((), ())),
                            preferred_element_type=jnp.float32)
            + b1_ref[...], 0.0)
        logits = lax.dot_general(h1, w2_ref[...], (((1,), (0,)), ((), ())),
                                 preferred_element_type=jnp.float32)
        logits = logits + b2_ref[...]
        z = logits - logits.max(axis=1, keepdims=True)
        ez = jnp.exp(z)
        out_ref[...] = ez / ez.sum(axis=1, keepdims=True)


def _stage_c(acc, gid2, w1t, b1r, w2t, b2r):
    return pl.pallas_call(
        _c_body,
        grid=(N_PAD // _BC,),
        in_specs=[
            pl.BlockSpec((H, _BC, FX), lambda i: (0, i, 0)),
            pl.BlockSpec((1, 1, _BC), lambda i: (i, 0, 0)),
            pl.BlockSpec((F, F // 2), lambda i: (0, 0)),
            pl.BlockSpec((1, F // 2), lambda i: (0, 0)),
            pl.BlockSpec((F // 2, CLS), lambda i: (0, 0)),
            pl.BlockSpec((1, CLS), lambda i: (0, 0)),
        ],
        out_specs=pl.BlockSpec((G, CLS), lambda i: (0, 0)),
        out_shape=jax.ShapeDtypeStruct((G, CLS), jnp.float32),
        scratch_shapes=[
            pltpu.VMEM((G, F), jnp.float32),
            pltpu.VMEM((G, F), jnp.float32),
        ],
    )(acc, gid2, w1t, b1r, w2t, b2r)


def kernel(x, edge_index, graph_ids, W_fc, attn_l, attn_r, W1, b1, W2, b2):
    w_r = W_fc.reshape(H, F, D)
    al3 = attn_l.reshape(H, 1, F)
    ar3 = attn_r.reshape(H, 1, F)
    featx, er2 = _stage_a(x, w_r, al3, ar3)

    pad = E_PAD - E
    srcp = jnp.concatenate([edge_index[0], jnp.zeros((pad,), jnp.int32)])
    dstp = jnp.concatenate([edge_index[1], jnp.zeros((pad,), jnp.int32)])
    zrows = jnp.zeros((ROWS_PT, FX), jnp.float32)
    acc = _stage_b(featx.reshape(H * N, FX), er2.reshape(H * N, 16),
                   srcp, dstp, zrows)

    gid2 = jnp.pad(graph_ids, (0, N_PAD - N),
                   constant_values=G).reshape(N_PAD // _BC, 1, _BC)
    return _stage_c(acc, gid2,
                    W1.T, b1.reshape(1, F // 2), W2.T, b2.reshape(1, CLS))


# speedup vs baseline: 17.6678x; 1.7151x over previous
import jax
import jax.numpy as jnp
from jax import lax
from jax.experimental import pallas as pl
from jax.experimental.pallas import tpu as pltpu
from jax.experimental.pallas import tpu_sc as plsc

N = 10000
E = 320000
D = 128
H = 5
F = 128
G = 16
CLS = 10
NEG_SLOPE = 0.2

FX = F + 16

NCORE = 2
NTILE = 16
C = 64
NCHUNK = 314
EPT = NCHUNK * C
E_PAD = EPT * NTILE
NCH_G = NTILE * NCHUNK
N_PAD = 10112
ROWS_PT = N_PAD // NTILE
NSLOT = 6


_BN = 1000


def _a_body(x_ref, w_ref, al_ref, ar_ref, featx_ref, er2_ref):
    xb = x_ref[...]
    w = w_ref[0]
    f = lax.dot_general(xb, w, (((1,), (1,)), ((), ())),
                        preferred_element_type=jnp.float32)
    al = al_ref[0, 0]
    ar = ar_ref[0, 0]
    el = (f * al[None, :]).sum(axis=1)
    er = (f * ar[None, :]).sum(axis=1)
    lane = lax.broadcasted_iota(jnp.int32, (_BN, 16), 1)
    featx_ref[0] = jnp.concatenate(
        [f, jnp.where(lane == 0, el[:, None], 0.0)], axis=1)
    er2_ref[0] = jnp.where(lane == 0, er[:, None], 0.0)


def _stage_a(x, w_r, al3, ar3):
    return pl.pallas_call(
        _a_body,
        grid=(N // _BN, H),
        in_specs=[
            pl.BlockSpec((_BN, D), lambda i, h: (i, 0)),
            pl.BlockSpec((1, F, D), lambda i, h: (h, 0, 0)),
            pl.BlockSpec((1, 1, F), lambda i, h: (h, 0, 0)),
            pl.BlockSpec((1, 1, F), lambda i, h: (h, 0, 0)),
        ],
        out_specs=[
            pl.BlockSpec((1, _BN, FX), lambda i, h: (h, i, 0)),
            pl.BlockSpec((1, _BN, 16), lambda i, h: (h, i, 0)),
        ],
        out_shape=[
            jax.ShapeDtypeStruct((H, N, FX), jnp.float32),
            jax.ShapeDtypeStruct((H, N, 16), jnp.float32),
        ],
    )(x, w_r, al3, ar3)


def _b_body(featx_hbm, er2_hbm, idxp_hbm, zrows_hbm, acc_out,
            idx_a, idx_b, fb_a, fb_b, erg_a, erg_b, acc_sh,
            si_a, si_b, sg_a, sg_b):
    c = lax.axis_index("c")
    s = lax.axis_index("s")
    rows0 = s * ROWS_PT

    def head_pass(h, k0, t_chunks, slot):
        bufa = (idx_a, fb_a, erg_a, si_a, sg_a)
        bufb = (idx_b, fb_b, erg_b, si_b, sg_b)

        def start_idx(k, buf):
            idxb, _, _, si, _ = buf
            pltpu.async_copy(idxp_hbm.at[h].at[s * NCHUNK + k], idxb, si)

        def wait_idx(buf):
            idxb, _, _, si, _ = buf
            pltpu.make_async_copy(idxp_hbm.at[0].at[0], idxb, si).wait()

        def start_g(buf):
            idxb, fb, ergb, _, sg = buf
            pltpu.async_copy(featx_hbm.at[idxb.at[0]], fb, sg)
            pltpu.async_copy(er2_hbm.at[idxb.at[1]], ergb, sg)

        def wait_g(buf):
            _, fb, ergb, _, sg = buf
            pltpu.make_async_copy(featx_hbm.at[pl.ds(0, C)], fb, sg).wait()
            pltpu.make_async_copy(er2_hbm.at[pl.ds(0, C)], ergb, sg).wait()

        lane = lax.iota(jnp.int32, 16)

        def comp(k, buf):
            idxb, fb, ergb, _, _ = buf
            ebase = s * EPT + k * C

            def _edge16(jj, _):
                for ll in range(16):
                    cc = jj * 16 + ll
                    ev = fb[cc, pl.ds(F, 16)] + ergb[cc, :]
                    ev = jnp.where(ev > 0, ev, NEG_SLOPE * ev)
                    ezv = jnp.exp(ev)
                    ezc = jnp.where(ebase + cc < E, ezv[0], 0.0)
                    for j in range(F // 16):
                        off = pl.ds(j * 16, 16)
                        fb[cc, off] = fb[cc, off] * ezc
                    fb[cc, pl.ds(F, 16)] = jnp.where(lane == 0, ezc, 0.0)
                return 0

            lax.fori_loop(0, C // 16, _edge16, 0)
            pltpu.sync_copy(fb, acc_sh.at[idxb.at[2]], add=True)

        pltpu.sync_copy(zrows_hbm, acc_sh.at[pl.ds(rows0, ROWS_PT)])
        plsc.subcore_barrier()

        start_idx(k0, bufa)
        wait_idx(bufa)
        start_g(bufa)
        start_idx(k0 + 1, bufb)

        def pair(p, _):
            ka = k0 + 2 * p
            wait_idx(bufb)
            wait_g(bufa)
            start_g(bufb)
            start_idx(ka + 2, bufa)
            comp(ka, bufa)
            wait_idx(bufa)
            wait_g(bufb)
            start_g(bufa)
            start_idx(ka + 3, bufb)
            comp(ka + 1, bufb)
            return 0

        npair = t_chunks // 2
        lax.fori_loop(0, npair - 1, pair, 0)
        klast = k0 + 2 * (npair - 1)
        if t_chunks % 2 == 0:
            wait_idx(bufb)
            wait_g(bufa)
            start_g(bufb)
            comp(klast, bufa)
            wait_g(bufb)
            comp(klast + 1, bufb)
        else:
            wait_idx(bufb)
            wait_g(bufa)
            start_g(bufb)
            start_idx(klast + 2, bufa)
            comp(klast, bufa)
            wait_idx(bufa)
            wait_g(bufb)
            start_g(bufa)
            comp(klast + 1, bufb)
            wait_g(bufa)
            comp(klast + 2, bufa)

        plsc.subcore_barrier()
        pltpu.sync_copy(acc_sh.at[pl.ds(rows0, ROWS_PT)],
                        acc_out.at[slot].at[pl.ds(rows0, ROWS_PT)])
        plsc.subcore_barrier()

    head_pass(2 * c, 0, NCHUNK, 2 * c)
    head_pass(2 * c + 1, 0, NCHUNK, 2 * c + 1)
    head_pass(jnp.int32(4), c * (NCHUNK // 2), NCHUNK // 2, 4 + c)


_stage_b = pl.kernel(
    _b_body,
    out_type=jax.ShapeDtypeStruct((NSLOT, N_PAD, FX), jnp.float32),
    mesh=plsc.VectorSubcoreMesh(core_axis_name="c", subcore_axis_name="s"),
    compiler_params=pltpu.CompilerParams(needs_layout_passes=False,
                                         use_tc_tiling_on_sc=False),
    scratch_types=[
        pltpu.VMEM((3, C), jnp.int32),
        pltpu.VMEM((3, C), jnp.int32),
        pltpu.VMEM((C, FX), jnp.float32),
        pltpu.VMEM((C, FX), jnp.float32),
        pltpu.VMEM((C, 16), jnp.float32),
        pltpu.VMEM((C, 16), jnp.float32),
        pltpu.VMEM_SHARED((N_PAD, FX), jnp.float32),
        pltpu.SemaphoreType.DMA,
        pltpu.SemaphoreType.DMA,
        pltpu.SemaphoreType.DMA,
        pltpu.SemaphoreType.DMA,
    ],
)


_BC = 632


def _c_body(acc_ref, gid_ref, w1_ref, b1_ref, w2_ref, b2_ref,
            out_ref, hg_ref, cnt_ref):
    i = pl.program_id(0)

    @pl.when(i == 0)
    def _():
        hg_ref[...] = jnp.zeros_like(hg_ref)
        cnt_ref[...] = jnp.zeros_like(cnt_ref)

    full = acc_ref[...]
    a = full[:, :, :F]
    sden = full[:, :, F]
    eps = 1e-30
    r03 = jnp.maximum(
        a[:4] / jnp.maximum(sden[:4], eps)[:, :, None], 0.0).sum(axis=0)
    a4 = a[4] + a[5]
    s4 = sden[4] + sden[5]
    r4 = jnp.maximum(a4 / jnp.maximum(s4, eps)[:, None], 0.0)
    hi = (r03 + r4) * (1.0 / H)
    g = gid_ref[0, 0, :]
    m = (g[None, :] == lax.broadcasted_iota(jnp.int32, (G, _BC), 0))
    m = m.astype(jnp.float32)
    hg_ref[...] += lax.dot_general(m, hi, (((1,), (0,)), ((), ())),
                                   preferred_element_type=jnp.float32)
    cnt_ref[...] += jnp.broadcast_to(m.sum(axis=1)[:, None], (G, F))

    @pl.when(i == pl.num_programs(0) - 1)
    def _():
        hg = hg_ref[...] / jnp.maximum(cnt_ref[...], 1.0)
        h1 = jnp.maximum(
            lax.dot_general(hg, w1_ref[...], (((1,), (0,)), ((), ())),
                            preferred_element_type=jnp.float32)
            + b1_ref[...], 0.0)
        logits = lax.dot_general(h1, w2_ref[...], (((1,), (0,)), ((), ())),
                                 preferred_element_type=jnp.float32)
        logits = logits + b2_ref[...]
        z = logits - logits.max(axis=1, keepdims=True)
        ez = jnp.exp(z)
        out_ref[...] = ez / ez.sum(axis=1, keepdims=True)


def _stage_c(acc, gid2, w1t, b1r, w2t, b2r):
    return pl.pallas_call(
        _c_body,
        grid=(N_PAD // _BC,),
        in_specs=[
            pl.BlockSpec((NSLOT, _BC, FX), lambda i: (0, i, 0)),
            pl.BlockSpec((1, 1, _BC), lambda i: (i, 0, 0)),
            pl.BlockSpec((F, F // 2), lambda i: (0, 0)),
            pl.BlockSpec((1, F // 2), lambda i: (0, 0)),
            pl.BlockSpec((F // 2, CLS), lambda i: (0, 0)),
            pl.BlockSpec((1, CLS), lambda i: (0, 0)),
        ],
        out_specs=pl.BlockSpec((G, CLS), lambda i: (0, 0)),
        out_shape=jax.ShapeDtypeStruct((G, CLS), jnp.float32),
        scratch_shapes=[
            pltpu.VMEM((G, F), jnp.float32),
            pltpu.VMEM((G, F), jnp.float32),
        ],
    )(acc, gid2, w1t, b1r, w2t, b2r)


def kernel(x, edge_index, graph_ids, W_fc, attn_l, attn_r, W1, b1, W2, b2):
    w_r = W_fc.reshape(H, F, D)
    al3 = attn_l.reshape(H, 1, F)
    ar3 = attn_r.reshape(H, 1, F)
    featx, er2 = _stage_a(x, w_r, al3, ar3)

    pad = E_PAD - E
    srcp = jnp.concatenate([edge_index[0], jnp.zeros((pad,), jnp.int32)])
    dstp = jnp.concatenate([edge_index[1], jnp.zeros((pad,), jnp.int32)])
    srcc = srcp.reshape(NCH_G, C)
    dstc = dstp.reshape(NCH_G, C)
    idxpack = jnp.stack(
        [jnp.stack([srcc + h * N, dstc + h * N, dstc], axis=1)
         for h in range(H)])

    zrows = jnp.zeros((ROWS_PT, FX), jnp.float32)
    acc = _stage_b(featx.reshape(H * N, FX), er2.reshape(H * N, 16),
                   idxpack, zrows)

    gid2 = jnp.pad(graph_ids, (0, N_PAD - N),
                   constant_values=G).reshape(N_PAD // _BC, 1, _BC)
    return _stage_c(acc, gid2,
                    W1.T, b1.reshape(1, F // 2), W2.T, b2.reshape(1, CLS))


# speedup vs baseline: 27.7577x; 1.5711x over previous
import jax
import jax.numpy as jnp
from jax import lax
from jax.experimental import pallas as pl
from jax.experimental.pallas import tpu as pltpu
from jax.experimental.pallas import tpu_sc as plsc

N = 10000
E = 320000
D = 128
H = 5
F = 128
G = 16
CLS = 10
NEG_SLOPE = 0.2

FX = F + 16

NCORE = 2
NTILE = 16
C = 64
NCHUNK = 314
EPT = NCHUNK * C
E_PAD = EPT * NTILE
NCH_G = NTILE * NCHUNK
N_PAD = 10112
ROWS_PT = N_PAD // NTILE
NSLOT = 6


_BN = 1000


def _a_body(x_ref, w_ref, al_ref, ar_ref, featx_ref, er2_ref):
    xb = x_ref[...]
    w = w_ref[0]
    f = lax.dot_general(xb, w, (((1,), (1,)), ((), ())),
                        preferred_element_type=jnp.float32)
    al = al_ref[0, 0]
    ar = ar_ref[0, 0]
    el = (f * al[None, :]).sum(axis=1)
    er = (f * ar[None, :]).sum(axis=1)
    lane = lax.broadcasted_iota(jnp.int32, (_BN, 16), 1)
    featx_ref[0] = jnp.concatenate(
        [f, jnp.where(lane == 0, el[:, None], 0.0)], axis=1)
    er2_ref[0] = jnp.where(lane == 0, er[:, None], 0.0)


def _stage_a(x, w_r, al3, ar3):
    return pl.pallas_call(
        _a_body,
        grid=(N // _BN, H),
        in_specs=[
            pl.BlockSpec((_BN, D), lambda i, h: (i, 0)),
            pl.BlockSpec((1, F, D), lambda i, h: (h, 0, 0)),
            pl.BlockSpec((1, 1, F), lambda i, h: (h, 0, 0)),
            pl.BlockSpec((1, 1, F), lambda i, h: (h, 0, 0)),
        ],
        out_specs=[
            pl.BlockSpec((1, _BN, FX), lambda i, h: (h, i, 0)),
            pl.BlockSpec((1, _BN, 16), lambda i, h: (h, i, 0)),
        ],
        out_shape=[
            jax.ShapeDtypeStruct((H, N, FX), jnp.float32),
            jax.ShapeDtypeStruct((H, N, 16), jnp.float32),
        ],
    )(x, w_r, al3, ar3)


def _b_body(featx_hbm, er2_hbm, idxp_hbm, zrows_hbm, acc_out,
            idx_a, idx_b, fb_a, fb_b, erg_a, erg_b, sx_a, sx_b, acc_sh,
            si_a, si_b, sg_a, sg_b, ss_a, ss_b):
    c = lax.axis_index("c")
    s = lax.axis_index("s")
    rows0 = s * ROWS_PT

    def head_pass(h, k0, t_chunks, slot):
        bufa = (idx_a, fb_a, erg_a, sx_a, si_a, sg_a, ss_a)
        bufb = (idx_b, fb_b, erg_b, sx_b, si_b, sg_b, ss_b)

        def start_idx(k, buf):
            idxb, si = buf[0], buf[4]
            pltpu.async_copy(idxp_hbm.at[h].at[s * NCHUNK + k], idxb, si)

        def wait_idx(buf):
            idxb, si = buf[0], buf[4]
            pltpu.make_async_copy(idxp_hbm.at[0].at[0], idxb, si).wait()

        def start_g(buf):
            idxb, fb, ergb, sg = buf[0], buf[1], buf[2], buf[5]
            pltpu.async_copy(featx_hbm.at[idxb.at[0]], fb, sg)
            pltpu.async_copy(er2_hbm.at[idxb.at[1]], ergb, sg)

        def wait_g(buf):
            fb, ergb, sg = buf[1], buf[2], buf[5]
            pltpu.make_async_copy(featx_hbm.at[pl.ds(0, C)], fb, sg).wait()
            pltpu.make_async_copy(er2_hbm.at[pl.ds(0, C)], ergb, sg).wait()

        lane = lax.iota(jnp.int32, 16)
        col_el = jnp.full((16,), F, jnp.int32)
        col_0 = jnp.zeros((16,), jnp.int32)

        def comp(k, buf):
            idxb, fb, ergb, sidx, ss = buf[0], buf[1], buf[2], buf[3], buf[6]
            ebase = s * EPT + k * C
            for q in range(C // 16):
                off = pl.ds(q * 16, 16)
                sidx[off] = idxb[2, off]

            def _edge16(jj, _):
                cc16 = jj * 16 + lane
                elv = plsc.load_gather(fb, [cc16, col_el])
                erv = plsc.load_gather(ergb, [cc16, col_0])
                ev = elv + erv
                ev = jnp.where(ev > 0, ev, NEG_SLOPE * ev)
                ezv = jnp.exp(ev)
                ezv = jnp.where(ebase + jj * 16 + lane < E, ezv, 0.0)
                plsc.store_scatter(fb, [cc16, col_el], ezv)
                for ll in range(16):
                    ezc = ezv[ll]
                    cc = jj * 16 + ll
                    for j in range(F // 16):
                        off = pl.ds(j * 16, 16)
                        fb[cc, off] = fb[cc, off] * ezc
                return 0

            lax.fori_loop(0, C // 16, _edge16, 0)
            pltpu.async_copy(fb, acc_sh.at[sidx], ss, add=True)

        def wait_sc(buf):
            fb, sidx, ss = buf[1], buf[3], buf[6]
            pltpu.make_async_copy(fb, acc_sh.at[sidx], ss).wait()

        pltpu.sync_copy(zrows_hbm, acc_sh.at[pl.ds(rows0, ROWS_PT)])
        plsc.subcore_barrier()

        start_idx(k0, bufa)
        wait_idx(bufa)
        start_g(bufa)
        start_idx(k0 + 1, bufb)

        wait_idx(bufb)
        wait_g(bufa)
        start_g(bufb)
        comp(k0, bufa)
        start_idx(k0 + 2, bufa)
        wait_g(bufb)
        wait_sc(bufa)
        wait_idx(bufa)
        start_g(bufa)
        comp(k0 + 1, bufb)
        start_idx(k0 + 3, bufb)

        def pair(p, _):
            ka = k0 + 2 * p
            wait_idx(bufb)
            wait_g(bufa)
            wait_sc(bufb)
            start_g(bufb)
            comp(ka, bufa)
            start_idx(ka + 2, bufa)
            wait_g(bufb)
            wait_sc(bufa)
            wait_idx(bufa)
            start_g(bufa)
            comp(ka + 1, bufb)
            start_idx(ka + 3, bufb)
            return 0

        npair = t_chunks // 2
        lax.fori_loop(1, npair - 1, pair, 0)
        klast = k0 + 2 * (npair - 1)
        if t_chunks % 2 == 0:
            wait_idx(bufb)
            wait_g(bufa)
            wait_sc(bufb)
            start_g(bufb)
            comp(klast, bufa)
            wait_g(bufb)
            wait_sc(bufa)
            comp(klast + 1, bufb)
            wait_sc(bufb)
        else:
            wait_idx(bufb)
            wait_g(bufa)
            wait_sc(bufb)
            start_g(bufb)
            comp(klast, bufa)
            start_idx(klast + 2, bufa)
            wait_g(bufb)
            wait_sc(bufa)
            wait_idx(bufa)
            start_g(bufa)
            comp(klast + 1, bufb)
            wait_g(bufa)
            wait_sc(bufb)
            comp(klast + 2, bufa)
            wait_sc(bufa)

        plsc.subcore_barrier()
        pltpu.sync_copy(acc_sh.at[pl.ds(rows0, ROWS_PT)],
                        acc_out.at[slot].at[pl.ds(rows0, ROWS_PT)])
        plsc.subcore_barrier()

    head_pass(2 * c, 0, NCHUNK, 2 * c)
    head_pass(2 * c + 1, 0, NCHUNK, 2 * c + 1)
    head_pass(jnp.int32(4), c * (NCHUNK // 2), NCHUNK // 2, 4 + c)


_stage_b = pl.kernel(
    _b_body,
    out_type=jax.ShapeDtypeStruct((NSLOT, N_PAD, FX), jnp.float32),
    mesh=plsc.VectorSubcoreMesh(core_axis_name="c", subcore_axis_name="s"),
    compiler_params=pltpu.CompilerParams(needs_layout_passes=False,
                                         use_tc_tiling_on_sc=False),
    scratch_types=[
        pltpu.VMEM((3, C), jnp.int32),
        pltpu.VMEM((3, C), jnp.int32),
        pltpu.VMEM((C, FX), jnp.float32),
        pltpu.VMEM((C, FX), jnp.float32),
        pltpu.VMEM((C, 16), jnp.float32),
        pltpu.VMEM((C, 16), jnp.float32),
        pltpu.VMEM((C,), jnp.int32),
        pltpu.VMEM((C,), jnp.int32),
        pltpu.VMEM_SHARED((N_PAD, FX), jnp.float32),
        pltpu.SemaphoreType.DMA,
        pltpu.SemaphoreType.DMA,
        pltpu.SemaphoreType.DMA,
        pltpu.SemaphoreType.DMA,
        pltpu.SemaphoreType.DMA,
        pltpu.SemaphoreType.DMA,
    ],
)


_BC = 632


def _c_body(acc_ref, gid_ref, w1_ref, b1_ref, w2_ref, b2_ref,
            out_ref, hg_ref, cnt_ref):
    i = pl.program_id(0)

    @pl.when(i == 0)
    def _():
        hg_ref[...] = jnp.zeros_like(hg_ref)
        cnt_ref[...] = jnp.zeros_like(cnt_ref)

    full = acc_ref[...]
    a = full[:, :, :F]
    sden = full[:, :, F]
    eps = 1e-30
    r03 = jnp.maximum(
        a[:4] / jnp.maximum(sden[:4], eps)[:, :, None], 0.0).sum(axis=0)
    a4 = a[4] + a[5]
    s4 = sden[4] + sden[5]
    r4 = jnp.maximum(a4 / jnp.maximum(s4, eps)[:, None], 0.0)
    hi = (r03 + r4) * (1.0 / H)
    g = gid_ref[0, 0, :]
    m = (g[None, :] == lax.broadcasted_iota(jnp.int32, (G, _BC), 0))
    m = m.astype(jnp.float32)
    hg_ref[...] += lax.dot_general(m, hi, (((1,), (0,)), ((), ())),
                                   preferred_element_type=jnp.float32)
    cnt_ref[...] += jnp.broadcast_to(m.sum(axis=1)[:, None], (G, F))

    @pl.when(i == pl.num_programs(0) - 1)
    def _():
        hg = hg_ref[...] / jnp.maximum(cnt_ref[...], 1.0)
        h1 = jnp.maximum(
            lax.dot_general(hg, w1_ref[...], (((1,), (0,)), ((), ())),
                            preferred_element_type=jnp.float32)
            + b1_ref[...], 0.0)
        logits = lax.dot_general(h1, w2_ref[...], (((1,), (0,)), ((), ())),
                                 preferred_element_type=jnp.float32)
        logits = logits + b2_ref[...]
        z = logits - logits.max(axis=1, keepdims=True)
        ez = jnp.exp(z)
        out_ref[...] = ez / ez.sum(axis=1, keepdims=True)


def _stage_c(acc, gid2, w1t, b1r, w2t, b2r):
    return pl.pallas_call(
        _c_body,
        grid=(N_PAD // _BC,),
        in_specs=[
            pl.BlockSpec((NSLOT, _BC, FX), lambda i: (0, i, 0)),
            pl.BlockSpec((1, 1, _BC), lambda i: (i, 0, 0)),
            pl.BlockSpec((F, F // 2), lambda i: (0, 0)),
            pl.BlockSpec((1, F // 2), lambda i: (0, 0)),
            pl.BlockSpec((F // 2, CLS), lambda i: (0, 0)),
            pl.BlockSpec((1, CLS), lambda i: (0, 0)),
        ],
        out_specs=pl.BlockSpec((G, CLS), lambda i: (0, 0)),
        out_shape=jax.ShapeDtypeStruct((G, CLS), jnp.float32),
        scratch_shapes=[
            pltpu.VMEM((G, F), jnp.float32),
            pltpu.VMEM((G, F), jnp.float32),
        ],
    )(acc, gid2, w1t, b1r, w2t, b2r)


def kernel(x, edge_index, graph_ids, W_fc, attn_l, attn_r, W1, b1, W2, b2):
    w_r = W_fc.reshape(H, F, D)
    al3 = attn_l.reshape(H, 1, F)
    ar3 = attn_r.reshape(H, 1, F)
    featx, er2 = _stage_a(x, w_r, al3, ar3)

    pad = E_PAD - E
    srcp = jnp.concatenate([edge_index[0], jnp.zeros((pad,), jnp.int32)])
    dstp = jnp.concatenate([edge_index[1], jnp.zeros((pad,), jnp.int32)])
    srcc = srcp.reshape(NCH_G, C)
    dstc = dstp.reshape(NCH_G, C)
    idxpack = jnp.stack(
        [jnp.stack([srcc + h * N, dstc + h * N, dstc], axis=1)
         for h in range(H)])

    zrows = jnp.zeros((ROWS_PT, FX), jnp.float32)
    acc = _stage_b(featx.reshape(H * N, FX), er2.reshape(H * N, 16),
                   idxpack, zrows)

    gid2 = jnp.pad(graph_ids, (0, N_PAD - N),
                   constant_values=G).reshape(N_PAD // _BC, 1, _BC)
    return _stage_c(acc, gid2,
                    W1.T, b1.reshape(1, F // 2), W2.T, b2.reshape(1, CLS))


# speedup vs baseline: 30.2076x; 1.0883x over previous
import jax
import jax.numpy as jnp
from jax import lax
from jax.experimental import pallas as pl
from jax.experimental.pallas import tpu as pltpu
from jax.experimental.pallas import tpu_sc as plsc

N = 10000
E = 320000
D = 128
H = 5
F = 128
G = 16
CLS = 10
NEG_SLOPE = 0.2

FX = F + 16

NCORE = 2
NTILE = 16
C = 64
NCHUNK = 314
EPT = NCHUNK * C
E_PAD = EPT * NTILE
NCH_G = NTILE * NCHUNK
N_PAD = 10112
ROWS_PT = N_PAD // NTILE
NSLOT = 6


_BN = 1000


def _a_body(x_ref, w_ref, al_ref, ar_ref, featx_ref, er2_ref):
    xb = x_ref[...]
    w = w_ref[0]
    f = lax.dot_general(xb, w, (((1,), (1,)), ((), ())),
                        preferred_element_type=jnp.float32)
    al = al_ref[0, 0]
    ar = ar_ref[0, 0]
    el = (f * al[None, :]).sum(axis=1)
    er = (f * ar[None, :]).sum(axis=1)
    lane = lax.broadcasted_iota(jnp.int32, (_BN, 16), 1)
    featx_ref[0] = jnp.concatenate(
        [f, jnp.where(lane == 0, el[:, None], 0.0)], axis=1)
    er2_ref[0] = jnp.where(lane == 0, er[:, None], 0.0)


def _stage_a(x, w_r, al3, ar3):
    return pl.pallas_call(
        _a_body,
        grid=(N // _BN, H),
        in_specs=[
            pl.BlockSpec((_BN, D), lambda i, h: (i, 0)),
            pl.BlockSpec((1, F, D), lambda i, h: (h, 0, 0)),
            pl.BlockSpec((1, 1, F), lambda i, h: (h, 0, 0)),
            pl.BlockSpec((1, 1, F), lambda i, h: (h, 0, 0)),
        ],
        out_specs=[
            pl.BlockSpec((1, _BN, FX), lambda i, h: (h, i, 0)),
            pl.BlockSpec((1, _BN, 16), lambda i, h: (h, i, 0)),
        ],
        out_shape=[
            jax.ShapeDtypeStruct((H, N, FX), jnp.float32),
            jax.ShapeDtypeStruct((H, N, 16), jnp.float32),
        ],
    )(x, w_r, al3, ar3)


def _b_body(featx_hbm, er2_hbm, idxp_hbm, zrows_hbm, acc_out,
            idx_a, idx_b, fb_a, fb_b, erg_a, erg_b, sx_a, sx_b, acc_sh,
            si_a, si_b, sg_a, sg_b, ss_a, ss_b):
    c = lax.axis_index("c")
    s = lax.axis_index("s")
    rows0 = s * ROWS_PT

    def head_pass(h, k0, t_chunks, slot):
        bufa = (idx_a, fb_a, erg_a, sx_a, si_a, sg_a, ss_a)
        bufb = (idx_b, fb_b, erg_b, sx_b, si_b, sg_b, ss_b)

        def start_idx(k, buf):
            idxb, si = buf[0], buf[4]
            pltpu.async_copy(idxp_hbm.at[s * NCHUNK + k], idxb, si)

        def wait_idx(buf):
            idxb, si = buf[0], buf[4]
            pltpu.make_async_copy(idxp_hbm.at[0], idxb, si).wait()

        def tf(buf):
            idxb = buf[0]
            hN = h * N
            for q in range(C // 16):
                off = pl.ds(q * 16, 16)
                idxb[0, off] = idxb[0, off] + hN
                idxb[1, off] = idxb[1, off] + hN

        def start_g(buf):
            idxb, fb, ergb, sg = buf[0], buf[1], buf[2], buf[5]
            pltpu.async_copy(featx_hbm.at[idxb.at[0]], fb, sg)
            pltpu.async_copy(er2_hbm.at[idxb.at[1]], ergb, sg)

        def wait_g(buf):
            fb, ergb, sg = buf[1], buf[2], buf[5]
            pltpu.make_async_copy(featx_hbm.at[pl.ds(0, C)], fb, sg).wait()
            pltpu.make_async_copy(er2_hbm.at[pl.ds(0, C)], ergb, sg).wait()

        lane = lax.iota(jnp.int32, 16)
        col_el = jnp.full((16,), F, jnp.int32)
        col_0 = jnp.zeros((16,), jnp.int32)

        def comp(k, buf):
            idxb, fb, ergb, sidx, ss = buf[0], buf[1], buf[2], buf[3], buf[6]
            ebase = s * EPT + k * C
            for q in range(C // 16):
                off = pl.ds(q * 16, 16)
                sidx[off] = idxb[2, off]

            def _edge16(jj, _):
                cc16 = jj * 16 + lane
                elv = plsc.load_gather(fb, [cc16, col_el])
                erv = plsc.load_gather(ergb, [cc16, col_0])
                ev = elv + erv
                ev = jnp.where(ev > 0, ev, NEG_SLOPE * ev)
                ezv = jnp.exp(ev)
                ezv = jnp.where(ebase + jj * 16 + lane < E, ezv, 0.0)
                plsc.store_scatter(fb, [cc16, col_el], ezv)
                for ll in range(16):
                    ezc = ezv[ll]
                    cc = jj * 16 + ll
                    for j in range(F // 16):
                        off = pl.ds(j * 16, 16)
                        fb[cc, off] = fb[cc, off] * ezc
                return 0

            lax.fori_loop(0, C // 16, _edge16, 0)
            pltpu.async_copy(fb, acc_sh.at[sidx], ss, add=True)

        def wait_sc(buf):
            fb, sidx, ss = buf[1], buf[3], buf[6]
            pltpu.make_async_copy(fb, acc_sh.at[sidx], ss).wait()

        pltpu.sync_copy(zrows_hbm, acc_sh.at[pl.ds(rows0, ROWS_PT)])
        plsc.subcore_barrier()

        start_idx(k0, bufa)
        wait_idx(bufa)
        tf(bufa)
        start_g(bufa)
        start_idx(k0 + 1, bufb)

        wait_idx(bufb)
        tf(bufb)
        wait_g(bufa)
        start_g(bufb)
        comp(k0, bufa)
        start_idx(k0 + 2, bufa)
        wait_g(bufb)
        wait_sc(bufa)
        wait_idx(bufa)
        tf(bufa)
        start_g(bufa)
        comp(k0 + 1, bufb)
        start_idx(k0 + 3, bufb)

        def pair(p, _):
            ka = k0 + 2 * p
            wait_idx(bufb)
            tf(bufb)
            wait_g(bufa)
            wait_sc(bufb)
            start_g(bufb)
            comp(ka, bufa)
            start_idx(ka + 2, bufa)
            wait_g(bufb)
            wait_sc(bufa)
            wait_idx(bufa)
            tf(bufa)
            start_g(bufa)
            comp(ka + 1, bufb)
            start_idx(ka + 3, bufb)
            return 0

        npair = t_chunks // 2
        lax.fori_loop(1, npair - 1, pair, 0)
        klast = k0 + 2 * (npair - 1)
        if t_chunks % 2 == 0:
            wait_idx(bufb)
            tf(bufb)
            wait_g(bufa)
            wait_sc(bufb)
            start_g(bufb)
            comp(klast, bufa)
            wait_g(bufb)
            wait_sc(bufa)
            comp(klast + 1, bufb)
            wait_sc(bufb)
        else:
            wait_idx(bufb)
            tf(bufb)
            wait_g(bufa)
            wait_sc(bufb)
            start_g(bufb)
            comp(klast, bufa)
            start_idx(klast + 2, bufa)
            wait_g(bufb)
            wait_sc(bufa)
            wait_idx(bufa)
            tf(bufa)
            start_g(bufa)
            comp(klast + 1, bufb)
            wait_g(bufa)
            wait_sc(bufb)
            comp(klast + 2, bufa)
            wait_sc(bufa)

        plsc.subcore_barrier()
        pltpu.sync_copy(acc_sh.at[pl.ds(rows0, ROWS_PT)],
                        acc_out.at[slot].at[pl.ds(rows0, ROWS_PT)])
        plsc.subcore_barrier()

    head_pass(2 * c, 0, NCHUNK, 2 * c)
    head_pass(2 * c + 1, 0, NCHUNK, 2 * c + 1)
    head_pass(jnp.int32(4), c * (NCHUNK // 2), NCHUNK // 2, 4 + c)


_stage_b = pl.kernel(
    _b_body,
    out_type=jax.ShapeDtypeStruct((NSLOT, N_PAD, FX), jnp.float32),
    mesh=plsc.VectorSubcoreMesh(core_axis_name="c", subcore_axis_name="s"),
    compiler_params=pltpu.CompilerParams(needs_layout_passes=False,
                                         use_tc_tiling_on_sc=False),
    scratch_types=[
        pltpu.VMEM((3, C), jnp.int32),
        pltpu.VMEM((3, C), jnp.int32),
        pltpu.VMEM((C, FX), jnp.float32),
        pltpu.VMEM((C, FX), jnp.float32),
        pltpu.VMEM((C, 16), jnp.float32),
        pltpu.VMEM((C, 16), jnp.float32),
        pltpu.VMEM((C,), jnp.int32),
        pltpu.VMEM((C,), jnp.int32),
        pltpu.VMEM_SHARED((N_PAD, FX), jnp.float32),
        pltpu.SemaphoreType.DMA,
        pltpu.SemaphoreType.DMA,
        pltpu.SemaphoreType.DMA,
        pltpu.SemaphoreType.DMA,
        pltpu.SemaphoreType.DMA,
        pltpu.SemaphoreType.DMA,
    ],
)


_BC = 632


def _c_body(acc_ref, gid_ref, w1_ref, b1_ref, w2_ref, b2_ref,
            out_ref, hg_ref, cnt_ref):
    i = pl.program_id(0)

    @pl.when(i == 0)
    def _():
        hg_ref[...] = jnp.zeros_like(hg_ref)
        cnt_ref[...] = jnp.zeros_like(cnt_ref)

    full = acc_ref[...]
    a = full[:, :, :F]
    sden = full[:, :, F]
    eps = 1e-30
    r03 = jnp.maximum(
        a[:4] / jnp.maximum(sden[:4], eps)[:, :, None], 0.0).sum(axis=0)
    a4 = a[4] + a[5]
    s4 = sden[4] + sden[5]
    r4 = jnp.maximum(a4 / jnp.maximum(s4, eps)[:, None], 0.0)
    hi = (r03 + r4) * (1.0 / H)
    g = gid_ref[0, 0, :]
    m = (g[None, :] == lax.broadcasted_iota(jnp.int32, (G, _BC), 0))
    m = m.astype(jnp.float32)
    hg_ref[...] += lax.dot_general(m, hi, (((1,), (0,)), ((), ())),
                                   preferred_element_type=jnp.float32)
    cnt_ref[...] += jnp.broadcast_to(m.sum(axis=1)[:, None], (G, F))

    @pl.when(i == pl.num_programs(0) - 1)
    def _():
        hg = hg_ref[...] / jnp.maximum(cnt_ref[...], 1.0)
        h1 = jnp.maximum(
            lax.dot_general(hg, w1_ref[...], (((1,), (0,)), ((), ())),
                            preferred_element_type=jnp.float32)
            + b1_ref[...], 0.0)
        logits = lax.dot_general(h1, w2_ref[...], (((1,), (0,)), ((), ())),
                                 preferred_element_type=jnp.float32)
        logits = logits + b2_ref[...]
        z = logits - logits.max(axis=1, keepdims=True)
        ez = jnp.exp(z)
        out_ref[...] = ez / ez.sum(axis=1, keepdims=True)


def _stage_c(acc, gid2, w1t, b1r, w2t, b2r):
    return pl.pallas_call(
        _c_body,
        grid=(N_PAD // _BC,),
        in_specs=[
            pl.BlockSpec((NSLOT, _BC, FX), lambda i: (0, i, 0)),
            pl.BlockSpec((1, 1, _BC), lambda i: (i, 0, 0)),
            pl.BlockSpec((F, F // 2), lambda i: (0, 0)),
            pl.BlockSpec((1, F // 2), lambda i: (0, 0)),
            pl.BlockSpec((F // 2, CLS), lambda i: (0, 0)),
            pl.BlockSpec((1, CLS), lambda i: (0, 0)),
        ],
        out_specs=pl.BlockSpec((G, CLS), lambda i: (0, 0)),
        out_shape=jax.ShapeDtypeStruct((G, CLS), jnp.float32),
        scratch_shapes=[
            pltpu.VMEM((G, F), jnp.float32),
            pltpu.VMEM((G, F), jnp.float32),
        ],
    )(acc, gid2, w1t, b1r, w2t, b2r)


def kernel(x, edge_index, graph_ids, W_fc, attn_l, attn_r, W1, b1, W2, b2):
    w_r = W_fc.reshape(H, F, D)
    al3 = attn_l.reshape(H, 1, F)
    ar3 = attn_r.reshape(H, 1, F)
    featx, er2 = _stage_a(x, w_r, al3, ar3)

    srcc = edge_index[0].reshape(E // C, C)
    dstc = edge_index[1].reshape(E // C, C)
    idxpack = jnp.pad(jnp.stack([srcc, dstc, dstc], axis=1),
                      ((0, NCH_G - E // C), (0, 0), (0, 0)))

    zrows = jnp.zeros((ROWS_PT, FX), jnp.float32)
    acc = _stage_b(featx.reshape(H * N, FX), er2.reshape(H * N, 16),
                   idxpack, zrows)

    gid2 = jnp.pad(graph_ids, (0, N_PAD - N),
                   constant_values=G).reshape(N_PAD // _BC, 1, _BC)
    return _stage_c(acc, gid2,
                    W1.T, b1.reshape(1, F // 2), W2.T, b2.reshape(1, CLS))
